# baseline (speedup 1.0000x reference)
import functools
import jax
import jax.numpy as jnp
from jax import lax
from jax.experimental import pallas as pl
from jax.experimental.pallas import tpu as pltpu
from jax.experimental.pallas import tpu_sc as plsc

NC, NS = 2, 16
NW = NC * NS


def _make_sc_gather(Q, V, L, t0, Lsc):
    per_w = -(-Lsc // NW)
    assert per_w <= 16
    H = Q // 2
    nfull = Lsc // NW
    nextra = Lsc - nfull * NW
    mesh = plsc.VectorSubcoreMesh(core_axis_name="c", subcore_axis_name="s")

    scratch = [
        pltpu.VMEM((L + 32,), jnp.int32),
        pltpu.VMEM((Q, H, 128), jnp.float32),
        pltpu.VMEM((Q, Q), jnp.float32),
    ]

    @functools.partial(
        pl.kernel,
        out_type=jax.ShapeDtypeStruct((Lsc * Q, Q), jnp.float32),
        mesh=mesh,
        scratch_types=scratch,
        compiler_params=pltpu.CompilerParams(needs_layout_passes=False),
    )
    def gather(ap_hbm, idx_hbm, out_hbm, idx_v, buf, out_v):
        wid = lax.axis_index("s") * NC + lax.axis_index("c")
        count = jnp.where(wid < nextra, nfull + 1, nfull)
        base = nfull * wid + jnp.minimum(wid, nextra)
        pltpu.sync_copy(idx_hbm, idx_v.at[pl.ds(0, L)])
        chunk = idx_v[pl.ds(t0 + base, 16)]
        lanes = lax.iota(jnp.int32, 16)
        for k in range(per_w):
            @pl.when(k < count)
            def _():
                s = chunk[k]
                u = pl.multiple_of((s // 128) * 128, 128)
                sloc = s - u
                for h in range(2):
                    pltpu.sync_copy(
                        ap_hbm.at[:, pl.ds(h * H, H), pl.ds(u, 128)], buf)
                    for q1 in range(Q):
                        vec = plsc.load_gather(
                            buf,
                            [jnp.full((16,), q1, jnp.int32),
                             lanes,
                             jnp.full((16,), sloc, jnp.int32)])
                        out_v[q1, pl.ds(h * H, H)] = vec
                pltpu.sync_copy(
                    out_v, out_hbm.at[pl.ds((base + k) * Q, Q), :])

    return gather


def _make_tc_gather(Q, V, Ltc):
    SPS = 10
    assert Ltc % SPS == 0

    def body(x_ref, *refs):
        a_refs, out_ref = refs[:SPS], refs[SPS]
        t = pl.program_id(0)
        lane = lax.broadcasted_iota(jnp.int32, (Q, Q, 128), 2)
        for j in range(SPS):
            sloc = x_ref[SPS * t + j] % 128
            blk = a_refs[j][...]
            out_ref[j * Q:(j + 1) * Q, :] = jnp.sum(
                jnp.where(lane == sloc, blk, 0.0), axis=2)

    def mk_index_map(j):
        return lambda t, xref: (0, 0, xref[SPS * t + j] // 128)

    grid_spec = pltpu.PrefetchScalarGridSpec(
        num_scalar_prefetch=1,
        grid=(Ltc // SPS,),
        in_specs=[pl.BlockSpec((Q, Q, 128), mk_index_map(j))
                  for j in range(SPS)],
        out_specs=pl.BlockSpec((SPS * Q, Q), lambda t, xref: (t, 0)),
    )
    return pl.pallas_call(
        body,
        grid_spec=grid_spec,
        out_shape=jax.ShapeDtypeStruct((Ltc * Q, Q), jnp.float32),
    )


def _make_chain(L, Q, G, Ltc):
    T = L // G
    assert T * G == L and Ltc % T == 0
    Gtc = Ltc // T

    def body(gtc_ref, gsc_ref, init_ref, final_ref, out_ref, d_ref):
        r = lax.broadcasted_iota(jnp.int32, (G * Q, Q), 0)
        c = lax.broadcasted_iota(jnp.int32, (G * Q, Q), 1)
        d_ref[...] = jnp.where((r % Q) == c, 1.0, 0.0).astype(jnp.float32)

        def step(i, carry):
            for g in range(G):
                if g < Gtc:
                    a = gtc_ref[pl.ds((g * T + i) * Q, Q), :]
                else:
                    a = gsc_ref[pl.ds(((g - Gtc) * T + i) * Q, Q), :]
                d = d_ref[g * Q:(g + 1) * Q, :]
                d_ref[g * Q:(g + 1) * Q, :] = jnp.dot(
                    d, a, preferred_element_type=jnp.float32)
            return carry

        lax.fori_loop(0, T, step, 0)

        x = init_ref[...]
        for g in range(G):
            x = jnp.dot(x, d_ref[g * Q:(g + 1) * Q, :],
                        preferred_element_type=jnp.float32)
        out_ref[...] = jnp.sum(x * final_ref[...], keepdims=True)

    return pl.pallas_call(
        body,
        out_shape=jax.ShapeDtypeStruct((1, 1), jnp.float32),
        scratch_shapes=[pltpu.VMEM((G * Q, Q), jnp.float32)],
    )


def kernel(A, input, init, final):
    Q, V, _ = A.shape
    L = input.shape[0]
    Ltc = 110
    ap = jnp.transpose(A, (0, 2, 1))
    x = input.astype(jnp.int32)
    g_sc = _make_sc_gather(Q, V, L, Ltc, L - Ltc)(ap, x)
    g_tc = _make_tc_gather(Q, V, Ltc)(x, *([ap] * 10))
    y = _make_chain(L, Q, 40, Ltc)(
        g_tc, g_sc, init.reshape(1, Q), final.reshape(1, Q))
    return y.reshape(())


# speedup vs baseline: 1.0336x; 1.0336x over previous
import functools
import jax
import jax.numpy as jnp
from jax import lax
from jax.experimental import pallas as pl
from jax.experimental.pallas import tpu as pltpu
from jax.experimental.pallas import tpu_sc as plsc

NC, NS = 2, 16
NW = NC * NS


def _make_sc_gather(Q, V, L, t0, Lsc):
    per_w = -(-Lsc // NW)
    assert per_w <= 16
    H = Q // 2
    nfull = Lsc // NW
    nextra = Lsc - nfull * NW
    mesh = plsc.VectorSubcoreMesh(core_axis_name="c", subcore_axis_name="s")

    scratch = [
        pltpu.VMEM((L + 32,), jnp.int32),
        pltpu.VMEM((Q, H, 128), jnp.float32),
        pltpu.VMEM((Q, Q), jnp.float32),
    ]

    @functools.partial(
        pl.kernel,
        out_type=jax.ShapeDtypeStruct((Lsc * Q, Q), jnp.float32),
        mesh=mesh,
        scratch_types=scratch,
        compiler_params=pltpu.CompilerParams(needs_layout_passes=False),
    )
    def gather(ap_hbm, idx_hbm, out_hbm, idx_v, buf, out_v):
        wid = lax.axis_index("s") * NC + lax.axis_index("c")
        count = jnp.where(wid < nextra, nfull + 1, nfull)
        base = nfull * wid + jnp.minimum(wid, nextra)
        pltpu.sync_copy(idx_hbm, idx_v.at[pl.ds(0, L)])
        chunk = idx_v[pl.ds(t0 + base, 16)]
        lanes = lax.iota(jnp.int32, 16)
        for k in range(per_w):
            @pl.when(k < count)
            def _():
                s = chunk[k]
                u = pl.multiple_of((s // 128) * 128, 128)
                sloc = s - u
                for h in range(2):
                    pltpu.sync_copy(
                        ap_hbm.at[:, pl.ds(h * H, H), pl.ds(u, 128)], buf)
                    for q1 in range(Q):
                        vec = plsc.load_gather(
                            buf,
                            [jnp.full((16,), q1, jnp.int32),
                             lanes,
                             jnp.full((16,), sloc, jnp.int32)])
                        out_v[q1, pl.ds(h * H, H)] = vec
                pltpu.sync_copy(
                    out_v, out_hbm.at[pl.ds((base + k) * Q, Q), :])

    return gather


def _make_tc_gather(Q, V, Ltc):
    SPS = 10
    assert Ltc % SPS == 0

    def body(x_ref, *refs):
        a_refs, out_ref = refs[:SPS], refs[SPS]
        t = pl.program_id(0)
        lane = lax.broadcasted_iota(jnp.int32, (Q, Q, 128), 2)
        for j in range(SPS):
            sloc = x_ref[SPS * t + j] % 128
            blk = a_refs[j][...]
            out_ref[j * Q:(j + 1) * Q, :] = jnp.sum(
                jnp.where(lane == sloc, blk, 0.0), axis=2)

    def mk_index_map(j):
        return lambda t, xref: (0, 0, xref[SPS * t + j] // 128)

    grid_spec = pltpu.PrefetchScalarGridSpec(
        num_scalar_prefetch=1,
        grid=(Ltc // SPS,),
        in_specs=[pl.BlockSpec((Q, Q, 128), mk_index_map(j))
                  for j in range(SPS)],
        out_specs=pl.BlockSpec((SPS * Q, Q), lambda t, xref: (t, 0)),
    )
    return pl.pallas_call(
        body,
        grid_spec=grid_spec,
        out_shape=jax.ShapeDtypeStruct((Ltc * Q, Q), jnp.float32),
    )


def _make_chain(L, Q, G, Ltc):
    T = L // G
    assert T * G == L and Ltc % T == 0
    Gtc = Ltc // T

    def body(gtc_ref, gsc_ref, init_ref, final_ref, out_ref, d_ref):
        r = lax.broadcasted_iota(jnp.int32, (G * Q, Q), 0)
        c = lax.broadcasted_iota(jnp.int32, (G * Q, Q), 1)
        d_ref[...] = jnp.where((r % Q) == c, 1.0, 0.0).astype(jnp.float32)

        UNROLL = 2
        assert T % UNROLL == 0

        def step(i0, carry):
            for u in range(UNROLL):
                i = UNROLL * i0 + u
                for g in range(G):
                    if g < Gtc:
                        a = gtc_ref[pl.ds((g * T + i) * Q, Q), :]
                    else:
                        a = gsc_ref[pl.ds(((g - Gtc) * T + i) * Q, Q), :]
                    d = d_ref[g * Q:(g + 1) * Q, :]
                    d_ref[g * Q:(g + 1) * Q, :] = jnp.dot(
                        d, a, preferred_element_type=jnp.float32)
            return carry

        lax.fori_loop(0, T // UNROLL, step, 0)

        x = init_ref[...]
        for g in range(G):
            x = jnp.dot(x, d_ref[g * Q:(g + 1) * Q, :],
                        preferred_element_type=jnp.float32)
        out_ref[...] = jnp.sum(x * final_ref[...], keepdims=True)

    return pl.pallas_call(
        body,
        out_shape=jax.ShapeDtypeStruct((1, 1), jnp.float32),
        scratch_shapes=[pltpu.VMEM((G * Q, Q), jnp.float32)],
    )


def kernel(A, input, init, final):
    Q, V, _ = A.shape
    L = input.shape[0]
    Ltc = 110
    ap = jnp.transpose(A, (0, 2, 1))
    x = input.astype(jnp.int32)
    g_sc = _make_sc_gather(Q, V, L, Ltc, L - Ltc)(ap, x)
    g_tc = _make_tc_gather(Q, V, Ltc)(x, *([ap] * 10))
    y = _make_chain(L, Q, 20, Ltc)(
        g_tc, g_sc, init.reshape(1, Q), final.reshape(1, Q))
    return y.reshape(())
